# Initial kernel scaffold; baseline (speedup 1.0000x reference)
import jax
import jax.numpy as jnp
from jax.experimental import pallas as pl


def kernel(features, edge_index, W0, b0, W1, b1, Wb0, bb0, Wb1, bb1):
    raise NotImplementedError("write your pallas kernel here")


# speedup vs baseline: 8.4293x; 8.4293x over previous
import functools

import jax
import jax.numpy as jnp
from jax import lax
from jax.experimental import pallas as pl
from jax.experimental.pallas import tpu as pltpu
from jax.experimental.pallas import tpu_sc as plsc

NC = 2
NS = 16
NW = NC * NS
K = 128
PADR = 240


def _make_deg_kernel(T, CD):
    mesh = plsc.VectorSubcoreMesh(core_axis_name="c", subcore_axis_name="s")
    ept = T // NS

    @functools.partial(
        pl.kernel,
        mesh=mesh,
        out_type=jax.ShapeDtypeStruct((NC, T), jnp.float32),
        scratch_types=[
            pltpu.VMEM((CD, K), jnp.int32),
            pltpu.VMEM((K,), jnp.float32),
            pltpu.VMEM((ept,), jnp.float32),
            pltpu.VMEM_SHARED((T,), jnp.float32),
            pltpu.SemaphoreType.DMA,
        ],
    )
    def body(idx_hbm, zeros_hbm, out_hbm, idx_v, ones_v, zb_v, deg_sh, sem):
        cid = lax.axis_index("c")
        sid = lax.axis_index("s")
        wid = cid * NS + sid
        base = sid * ept
        for u in range(K // 16):
            ones_v[pl.ds(u * 16, 16)] = jnp.ones((16,), jnp.float32)
        pltpu.sync_copy(zeros_hbm, zb_v)
        pltpu.sync_copy(zb_v, deg_sh.at[pl.ds(base, ept)])
        pltpu.sync_copy(idx_hbm.at[wid], idx_v)
        plsc.subcore_barrier()

        def chunk(j, carry):
            pltpu.sync_copy(ones_v, deg_sh.at[idx_v.at[j]], add=True)
            return carry

        lax.fori_loop(0, CD, chunk, 0)
        plsc.subcore_barrier()
        pltpu.sync_copy(deg_sh.at[pl.ds(base, ept)],
                        out_hbm.at[cid, pl.ds(base, ept)])

    return body


def _make_pass_kernel(NPAD, D, CPW):
    mesh = plsc.VectorSubcoreMesh(core_axis_name="c", subcore_axis_name="s")
    rpt = NPAD // NS

    @functools.partial(
        pl.kernel,
        mesh=mesh,
        out_type=jax.ShapeDtypeStruct((NC, NPAD, D), jnp.float32),
        scratch_types=[
            pltpu.VMEM((CPW, K), jnp.int32),
            pltpu.VMEM((CPW, K), jnp.int32),
            pltpu.VMEM((K, D), jnp.float32),
            pltpu.VMEM_SHARED((NPAD, D), jnp.float32),
            pltpu.SemaphoreType.DMA,
        ],
    )
    def body(src_hbm, dst_hbm, feat_hbm, zeros_hbm, out_hbm,
             src_v, dst_v, rows_v, agg_sh, sem):
        cid = lax.axis_index("c")
        sid = lax.axis_index("s")
        wid = cid * NS + sid
        base = sid * rpt
        pltpu.sync_copy(zeros_hbm, rows_v)
        for z in range(rpt // K):
            pltpu.sync_copy(rows_v, agg_sh.at[pl.ds(base + z * K, K)])
        pltpu.sync_copy(src_hbm.at[wid], src_v)
        pltpu.sync_copy(dst_hbm.at[wid], dst_v)
        plsc.subcore_barrier()

        def chunk(j, carry):
            pltpu.async_copy(feat_hbm.at[src_v.at[j]], rows_v, sem).wait()
            pltpu.sync_copy(rows_v, agg_sh.at[dst_v.at[j]], add=True)
            return carry

        lax.fori_loop(0, CPW, chunk, 0)
        plsc.subcore_barrier()
        pltpu.sync_copy(agg_sh.at[pl.ds(base, rpt)],
                        out_hbm.at[cid, pl.ds(base, rpt)])

    return body


def _norms_call(degparts4, SUB):
    def body(p_ref, o_ref):
        p = p_ref[...]
        in_deg = p[0, 0] + p[1, 0]
        out_deg = p[0, 1] + p[1, 1]
        in_c = jnp.maximum(in_deg, 1.0)
        out_c = jnp.maximum(out_deg, 1.0)
        o_ref[0] = lax.rsqrt(out_c)
        o_ref[1] = lax.rsqrt(in_c)
        o_ref[2] = 1.0 / in_c

    return pl.pallas_call(
        body,
        out_shape=jax.ShapeDtypeStruct((3, SUB, 128), jnp.float32),
    )(degparts4)


def _mm_scale_call(x, w, s, RB):
    n, d = x.shape

    def body(x_ref, w_ref, s_ref, o_ref):
        o_ref[...] = jnp.dot(x_ref[...], w_ref[...],
                             preferred_element_type=jnp.float32) * s_ref[...]

    return pl.pallas_call(
        body,
        grid=(n // RB,),
        in_specs=[
            pl.BlockSpec((RB, d), lambda i: (i, 0)),
            pl.BlockSpec((d, d), lambda i: (0, 0)),
            pl.BlockSpec((RB, 1), lambda i: (i, 0)),
        ],
        out_specs=pl.BlockSpec((RB, d), lambda i: (i, 0)),
        out_shape=jax.ShapeDtypeStruct((n, d), jnp.float32),
    )(x, w, s)


def _mid_call(pa, pb, dstn, ninv, srcn, x, wb0, b0r, bb0r, w1, RB):
    n, d = x.shape

    def body(pa_ref, pb_ref, dn_ref, ni_ref, sn_ref, x_ref, wb_ref, b0_ref,
             bb_ref, w1_ref, h1_ref, f1_ref):
        agg = pa_ref[...] + pb_ref[...]
        conv = agg * dn_ref[...] + b0_ref[...]
        buf = jnp.dot(x_ref[...], wb_ref[...],
                      preferred_element_type=jnp.float32) + bb_ref[...]
        h1 = jnp.maximum(conv + ni_ref[...] * buf, 0.0)
        h1_ref[...] = h1
        f1_ref[...] = jnp.dot(h1, w1_ref[...],
                              preferred_element_type=jnp.float32) * sn_ref[...]

    col = pl.BlockSpec((RB, 1), lambda i: (i, 0))
    mat = pl.BlockSpec((RB, d), lambda i: (i, 0))
    wsp = pl.BlockSpec((d, d), lambda i: (0, 0))
    bsp = pl.BlockSpec((1, d), lambda i: (0, 0))
    return pl.pallas_call(
        body,
        grid=(n // RB,),
        in_specs=[mat, mat, col, col, col, mat, wsp, bsp, bsp, wsp],
        out_specs=[mat, mat],
        out_shape=[
            jax.ShapeDtypeStruct((n, d), jnp.float32),
            jax.ShapeDtypeStruct((n, d), jnp.float32),
        ],
    )(pa, pb, dstn, ninv, srcn, x, wb0, b0r, bb0r, w1)


def _final_call(pa, pb, dstn, ninv, x, h1, wb1a, wb1b, b1r, bb1r, RB):
    n, d = x.shape

    def body(pa_ref, pb_ref, dn_ref, ni_ref, x_ref, h1_ref, wa_ref, wb_ref,
             b1_ref, bb_ref, o_ref):
        agg = pa_ref[...] + pb_ref[...]
        conv = agg * dn_ref[...] + b1_ref[...]
        buf = (jnp.dot(x_ref[...], wa_ref[...],
                       preferred_element_type=jnp.float32)
               + jnp.dot(h1_ref[...], wb_ref[...],
                         preferred_element_type=jnp.float32) + bb_ref[...])
        o_ref[...] = conv + ni_ref[...] * buf

    col = pl.BlockSpec((RB, 1), lambda i: (i, 0))
    mat = pl.BlockSpec((RB, d), lambda i: (i, 0))
    wsp = pl.BlockSpec((d, d), lambda i: (0, 0))
    bsp = pl.BlockSpec((1, d), lambda i: (0, 0))
    return pl.pallas_call(
        body,
        grid=(n // RB,),
        in_specs=[mat, mat, col, col, mat, mat, wsp, wsp, bsp, bsp],
        out_specs=mat,
        out_shape=jax.ShapeDtypeStruct((n, d), jnp.float32),
    )(pa, pb, dstn, ninv, x, h1, wb1a, wb1b, b1r, bb1r)


def kernel(features, edge_index, W0, b0, W1, b1, Wb0, bb0, Wb1, bb1):
    N, D = features.shape
    E = edge_index.shape[1]
    NPAD = N + PADR
    src = edge_index[0]
    dst = edge_index[1]

    CPW = -(-E // (NW * K))
    E2 = NW * CPW * K
    pe = jnp.arange(E2 - E, dtype=jnp.int32)
    src_p = jnp.concatenate([src, pe % N]).reshape(NW, CPW, K)
    dst_p = jnp.concatenate([dst, N + pe % PADR]).reshape(NW, CPW, K)

    T = 2 * NPAD
    DE = 2 * E
    CD = -(-DE // (NW * K))
    pd = jnp.arange(NW * CD * K - DE, dtype=jnp.int32)
    degidx = jnp.concatenate([dst, src + NPAD, N + pd % PADR])
    degidx = degidx.reshape(NW, CD, K)

    zeros_e = jnp.zeros((T // NS,), jnp.float32)
    zeros_r = jnp.zeros((K, D), jnp.float32)

    degparts = _make_deg_kernel(T, CD)(degidx, zeros_e)
    SUB = NPAD // 128
    norms = _norms_call(degparts.reshape(NC, 2, SUB, 128), SUB)
    nr = norms.reshape(3, NPAD)
    srcn = nr[0, :N].reshape(N, 1)
    dstn = nr[1, :N].reshape(N, 1)
    ninv = nr[2, :N].reshape(N, 1)

    RB = 1000
    b0r, bb0r = b0.reshape(1, D), bb0.reshape(1, D)
    b1r, bb1r = b1.reshape(1, D), bb1.reshape(1, D)
    wb1a, wb1b = Wb1[:D], Wb1[D:]

    pass_fn = _make_pass_kernel(NPAD, D, CPW)

    feat0 = _mm_scale_call(features, W0, srcn, RB)
    parts0 = pass_fn(src_p, dst_p, feat0, zeros_r)
    h1, feat1 = _mid_call(parts0[0, :N], parts0[1, :N], dstn, ninv, srcn,
                          features, Wb0, b0r, bb0r, W1, RB)

    parts1 = pass_fn(src_p, dst_p, feat1, zeros_r)
    out = _final_call(parts1[0, :N], parts1[1, :N], dstn, ninv,
                      features, h1, wb1a, wb1b, b1r, bb1r, RB)
    return out


# speedup vs baseline: 10.5267x; 1.2488x over previous
import functools

import jax
import jax.numpy as jnp
from jax import lax
from jax.experimental import pallas as pl
from jax.experimental.pallas import tpu as pltpu
from jax.experimental.pallas import tpu_sc as plsc

NC = 2
NS = 16
NW = NC * NS
K = 128
PADR = 240
NBUF = 4
DH = 64


def _make_deg_kernel(T, CD):
    mesh = plsc.VectorSubcoreMesh(core_axis_name="c", subcore_axis_name="s")
    ept = T // NS

    @functools.partial(
        pl.kernel,
        mesh=mesh,
        out_type=jax.ShapeDtypeStruct((NC, T), jnp.float32),
        scratch_types=[
            pltpu.VMEM((CD, K), jnp.int32),
            pltpu.VMEM((K,), jnp.float32),
            pltpu.VMEM((ept,), jnp.float32),
            pltpu.VMEM_SHARED((T,), jnp.float32),
            pltpu.SemaphoreType.DMA,
        ],
    )
    def body(idx_hbm, zeros_hbm, out_hbm, idx_v, ones_v, zb_v, deg_sh, sem):
        cid = lax.axis_index("c")
        sid = lax.axis_index("s")
        wid = cid * NS + sid
        base = sid * ept
        for u in range(K // 16):
            ones_v[pl.ds(u * 16, 16)] = jnp.ones((16,), jnp.float32)
        pltpu.sync_copy(zeros_hbm, zb_v)
        pltpu.sync_copy(zb_v, deg_sh.at[pl.ds(base, ept)])
        pltpu.sync_copy(idx_hbm.at[wid], idx_v)
        plsc.subcore_barrier()

        def chunk(j, carry):
            pltpu.sync_copy(ones_v, deg_sh.at[idx_v.at[j]], add=True)
            return carry

        lax.fori_loop(0, CD, chunk, 0)
        plsc.subcore_barrier()
        pltpu.sync_copy(deg_sh.at[pl.ds(base, ept)],
                        out_hbm.at[cid, pl.ds(base, ept)])

    return body


def _make_pass_kernel(NPAD, CPW):
    mesh = plsc.VectorSubcoreMesh(core_axis_name="c", subcore_axis_name="s")
    rpt = NPAD // NS
    NG = CPW // NBUF

    @functools.partial(
        pl.kernel,
        mesh=mesh,
        out_type=jax.ShapeDtypeStruct((NC, NPAD, DH), jnp.float32),
        compiler_params=pltpu.CompilerParams(use_tc_tiling_on_sc=False),
        scratch_types=[
            pltpu.VMEM((CPW, K), jnp.int32),
            pltpu.VMEM((CPW, K), jnp.int32),
            pltpu.VMEM((NBUF, K, DH), jnp.float32),
            pltpu.VMEM_SHARED((NPAD, DH), jnp.float32),
            [pltpu.SemaphoreType.DMA] * NBUF,
            [pltpu.SemaphoreType.DMA] * NBUF,
        ],
    )
    def body(src_hbm, dst_hbm, feat_hbm, zeros_hbm, out_hbm,
             src_v, dst_v, rows_v, agg_sh, gsems, ssems):
        cid = lax.axis_index("c")
        sid = lax.axis_index("s")
        base = sid * rpt
        myfeat = feat_hbm.at[cid]
        pltpu.sync_copy(zeros_hbm, rows_v.at[0])
        for z in range(rpt // K):
            pltpu.sync_copy(rows_v.at[0], agg_sh.at[pl.ds(base + z * K, K)])
        pltpu.sync_copy(src_hbm.at[sid], src_v)
        pltpu.sync_copy(dst_hbm.at[sid], dst_v)
        plsc.subcore_barrier()

        for b in range(NBUF):
            pltpu.async_copy(myfeat.at[src_v.at[b]], rows_v.at[b], gsems[b])

        def group(g, carry):
            for b in range(NBUF):
                j = g * NBUF + b
                pltpu.make_async_copy(myfeat.at[src_v.at[j]],
                                      rows_v.at[b], gsems[b]).wait()
                pltpu.async_copy(rows_v.at[b], agg_sh.at[dst_v.at[j]],
                                 ssems[b], add=True)
            for b in range(NBUF):
                j2 = (g + 1) * NBUF + b
                pltpu.make_async_copy(rows_v.at[b], agg_sh.at[dst_v.at[0]],
                                      ssems[b]).wait()

                @pl.when(j2 < CPW)
                def _():
                    pltpu.async_copy(myfeat.at[src_v.at[j2]],
                                     rows_v.at[b], gsems[b])

            return carry

        lax.fori_loop(0, NG, group, 0)
        plsc.subcore_barrier()
        pltpu.sync_copy(agg_sh.at[pl.ds(base, rpt)],
                        out_hbm.at[cid, pl.ds(base, rpt)])

    return body


def _norms_call(degparts4, SUB):
    def body(p_ref, o_ref):
        p = p_ref[...]
        in_deg = p[0, 0] + p[1, 0]
        out_deg = p[0, 1] + p[1, 1]
        in_c = jnp.maximum(in_deg, 1.0)
        out_c = jnp.maximum(out_deg, 1.0)
        o_ref[0] = lax.rsqrt(out_c)
        o_ref[1] = lax.rsqrt(in_c)
        o_ref[2] = 1.0 / in_c

    return pl.pallas_call(
        body,
        out_shape=jax.ShapeDtypeStruct((3, SUB, 128), jnp.float32),
    )(degparts4)


def _mm_scale_call(x, w, s, RB):
    n, d = x.shape

    def body(x_ref, w_ref, s_ref, o_ref):
        r = jnp.dot(x_ref[...], w_ref[...],
                    preferred_element_type=jnp.float32) * s_ref[...]
        o_ref[0] = r[:, :DH]
        o_ref[1] = r[:, DH:]

    return pl.pallas_call(
        body,
        grid=(n // RB,),
        in_specs=[
            pl.BlockSpec((RB, d), lambda i: (i, 0)),
            pl.BlockSpec((d, d), lambda i: (0, 0)),
            pl.BlockSpec((RB, 1), lambda i: (i, 0)),
        ],
        out_specs=pl.BlockSpec((NC, RB, DH), lambda i: (0, i, 0)),
        out_shape=jax.ShapeDtypeStruct((NC, n, DH), jnp.float32),
    )(x, w, s)


def _mid_call(parts, dstn, ninv, srcn, x, wb0, b0r, bb0r, w1, RB):
    n, d = x.shape

    def body(p_ref, dn_ref, ni_ref, sn_ref, x_ref, wb_ref, b0_ref,
             bb_ref, w1_ref, h1_ref, f1_ref):
        agg = jnp.concatenate([p_ref[0], p_ref[1]], axis=-1)
        conv = agg * dn_ref[...] + b0_ref[...]
        buf = jnp.dot(x_ref[...], wb_ref[...],
                      preferred_element_type=jnp.float32) + bb_ref[...]
        h1 = jnp.maximum(conv + ni_ref[...] * buf, 0.0)
        h1_ref[...] = h1
        f1 = jnp.dot(h1, w1_ref[...],
                     preferred_element_type=jnp.float32) * sn_ref[...]
        f1_ref[0] = f1[:, :DH]
        f1_ref[1] = f1[:, DH:]

    col = pl.BlockSpec((RB, 1), lambda i: (i, 0))
    mat = pl.BlockSpec((RB, d), lambda i: (i, 0))
    wsp = pl.BlockSpec((d, d), lambda i: (0, 0))
    bsp = pl.BlockSpec((1, d), lambda i: (0, 0))
    psp = pl.BlockSpec((NC, RB, DH), lambda i: (0, i, 0))
    return pl.pallas_call(
        body,
        grid=(n // RB,),
        in_specs=[psp, col, col, col, mat, wsp, bsp, bsp, wsp],
        out_specs=[mat, psp],
        out_shape=[
            jax.ShapeDtypeStruct((n, d), jnp.float32),
            jax.ShapeDtypeStruct((NC, n, DH), jnp.float32),
        ],
    )(parts, dstn, ninv, srcn, x, wb0, b0r, bb0r, w1)


def _final_call(parts, dstn, ninv, x, h1, wb1a, wb1b, b1r, bb1r, RB):
    n, d = x.shape

    def body(p_ref, dn_ref, ni_ref, x_ref, h1_ref, wa_ref, wb_ref,
             b1_ref, bb_ref, o_ref):
        agg = jnp.concatenate([p_ref[0], p_ref[1]], axis=-1)
        conv = agg * dn_ref[...] + b1_ref[...]
        buf = (jnp.dot(x_ref[...], wa_ref[...],
                       preferred_element_type=jnp.float32)
               + jnp.dot(h1_ref[...], wb_ref[...],
                         preferred_element_type=jnp.float32) + bb_ref[...])
        o_ref[...] = conv + ni_ref[...] * buf

    col = pl.BlockSpec((RB, 1), lambda i: (i, 0))
    mat = pl.BlockSpec((RB, d), lambda i: (i, 0))
    wsp = pl.BlockSpec((d, d), lambda i: (0, 0))
    bsp = pl.BlockSpec((1, d), lambda i: (0, 0))
    psp = pl.BlockSpec((NC, RB, DH), lambda i: (0, i, 0))
    return pl.pallas_call(
        body,
        grid=(n // RB,),
        in_specs=[psp, col, col, mat, mat, wsp, wsp, bsp, bsp],
        out_specs=mat,
        out_shape=jax.ShapeDtypeStruct((n, d), jnp.float32),
    )(parts, dstn, ninv, x, h1, wb1a, wb1b, b1r, bb1r)


def kernel(features, edge_index, W0, b0, W1, b1, Wb0, bb0, Wb1, bb1):
    N, D = features.shape
    E = edge_index.shape[1]
    NPAD = N + PADR
    src = edge_index[0]
    dst = edge_index[1]

    CPW = -(-E // (NS * K))
    CPW = -(-CPW // NBUF) * NBUF
    E2 = NS * CPW * K
    pe = jnp.arange(E2 - E, dtype=jnp.int32)
    src_p = jnp.concatenate([src, pe % N]).reshape(NS, CPW, K)
    dst_p = jnp.concatenate([dst, N + pe % PADR]).reshape(NS, CPW, K)

    T = 2 * NPAD
    DE = 2 * E
    CD = -(-DE // (NW * K))
    pd = jnp.arange(NW * CD * K - DE, dtype=jnp.int32)
    degidx = jnp.concatenate([dst, src + NPAD, N + pd % PADR])
    degidx = degidx.reshape(NW, CD, K)

    zeros_e = jnp.zeros((T // NS,), jnp.float32)
    zeros_r = jnp.zeros((K, DH), jnp.float32)

    degparts = _make_deg_kernel(T, CD)(degidx, zeros_e)
    SUB = NPAD // 128
    norms = _norms_call(degparts.reshape(NC, 2, SUB, 128), SUB)
    nr = norms.reshape(3, NPAD)
    srcn = nr[0, :N].reshape(N, 1)
    dstn = nr[1, :N].reshape(N, 1)
    ninv = nr[2, :N].reshape(N, 1)

    RB = 1000
    b0r, bb0r = b0.reshape(1, D), bb0.reshape(1, D)
    b1r, bb1r = b1.reshape(1, D), bb1.reshape(1, D)
    wb1a, wb1b = Wb1[:D], Wb1[D:]

    pass_fn = _make_pass_kernel(NPAD, CPW)

    feat0 = _mm_scale_call(features, W0, srcn, RB)
    parts0 = pass_fn(src_p, dst_p, feat0, zeros_r)
    h1, feat1 = _mid_call(parts0[:, :N], dstn, ninv, srcn,
                          features, Wb0, b0r, bb0r, W1, RB)

    parts1 = pass_fn(src_p, dst_p, feat1, zeros_r)
    out = _final_call(parts1[:, :N], dstn, ninv,
                      features, h1, wb1a, wb1b, b1r, bb1r, RB)
    return out


# speedup vs baseline: 10.8720x; 1.0328x over previous
import functools

import jax
import jax.numpy as jnp
from jax import lax
from jax.experimental import pallas as pl
from jax.experimental.pallas import tpu as pltpu
from jax.experimental.pallas import tpu_sc as plsc

NC = 2
NS = 16
NW = NC * NS
K = 128
PADR = 240
NBUF = 4
DH = 64


def _make_deg_kernel(T, CD):
    mesh = plsc.VectorSubcoreMesh(core_axis_name="c", subcore_axis_name="s")
    ept = T // NS

    @functools.partial(
        pl.kernel,
        mesh=mesh,
        out_type=jax.ShapeDtypeStruct((NC, T), jnp.float32),
        scratch_types=[
            pltpu.VMEM((CD, K), jnp.int32),
            pltpu.VMEM((K,), jnp.float32),
            pltpu.VMEM((ept,), jnp.float32),
            pltpu.VMEM_SHARED((T,), jnp.float32),
            pltpu.SemaphoreType.DMA,
        ],
    )
    def body(idx_hbm, zeros_hbm, out_hbm, idx_v, ones_v, zb_v, deg_sh, sem):
        cid = lax.axis_index("c")
        sid = lax.axis_index("s")
        wid = cid * NS + sid
        base = sid * ept
        for u in range(K // 16):
            ones_v[pl.ds(u * 16, 16)] = jnp.ones((16,), jnp.float32)
        pltpu.sync_copy(zeros_hbm, zb_v)
        pltpu.sync_copy(zb_v, deg_sh.at[pl.ds(base, ept)])
        pltpu.sync_copy(idx_hbm.at[wid], idx_v)
        plsc.subcore_barrier()

        def chunk(j, carry):
            pltpu.sync_copy(ones_v, deg_sh.at[idx_v.at[j]], add=True)
            return carry

        lax.fori_loop(0, CD, chunk, 0)
        plsc.subcore_barrier()
        pltpu.sync_copy(deg_sh.at[pl.ds(base, ept)],
                        out_hbm.at[cid, pl.ds(base, ept)])

    return body


def _make_pass_kernel(NPAD, CPW):
    mesh = plsc.VectorSubcoreMesh(core_axis_name="c", subcore_axis_name="s")
    rpt = NPAD // NS
    NG = CPW // NBUF

    @functools.partial(
        pl.kernel,
        mesh=mesh,
        out_type=jax.ShapeDtypeStruct((NC, NPAD, DH), jnp.float32),
        compiler_params=pltpu.CompilerParams(use_tc_tiling_on_sc=False),
        scratch_types=[
            pltpu.VMEM((CPW, K), jnp.int32),
            pltpu.VMEM((CPW, K), jnp.int32),
            pltpu.VMEM((NBUF, K, DH), jnp.float32),
            pltpu.VMEM_SHARED((NPAD, DH), jnp.float32),
            [pltpu.SemaphoreType.DMA] * NBUF,
            [pltpu.SemaphoreType.DMA] * NBUF,
        ],
    )
    def body(src_hbm, dst_hbm, feat_hbm, zeros_hbm, out_hbm,
             src_v, dst_v, rows_v, agg_sh, gsems, ssems):
        cid = lax.axis_index("c")
        sid = lax.axis_index("s")
        base = sid * rpt
        myfeat = feat_hbm.at[cid]
        pltpu.sync_copy(zeros_hbm, rows_v.at[0])
        for z in range(rpt // K):
            pltpu.sync_copy(rows_v.at[0], agg_sh.at[pl.ds(base + z * K, K)])
        pltpu.sync_copy(src_hbm.at[sid], src_v)
        pltpu.sync_copy(dst_hbm.at[sid], dst_v)
        plsc.subcore_barrier()

        for b in range(NBUF):
            pltpu.async_copy(myfeat.at[src_v.at[b]], rows_v.at[b], gsems[b])

        def group(g, carry):
            for b in range(NBUF):
                j = g * NBUF + b
                pltpu.make_async_copy(myfeat.at[src_v.at[j]],
                                      rows_v.at[b], gsems[b]).wait()
                pltpu.async_copy(rows_v.at[b], agg_sh.at[dst_v.at[j]],
                                 ssems[b], add=True)
            for b in range(NBUF):
                j2 = (g + 1) * NBUF + b
                pltpu.make_async_copy(rows_v.at[b], agg_sh.at[dst_v.at[0]],
                                      ssems[b]).wait()

                @pl.when(j2 < CPW)
                def _():
                    pltpu.async_copy(myfeat.at[src_v.at[j2]],
                                     rows_v.at[b], gsems[b])

            return carry

        lax.fori_loop(0, NG, group, 0)
        plsc.subcore_barrier()
        pltpu.sync_copy(agg_sh.at[pl.ds(base, rpt)],
                        out_hbm.at[cid, pl.ds(base, rpt)])

    return body


def _mm_scale_call(x, w, outp, NPAD, RB):
    n, d = x.shape

    def body(x_ref, w_ref, op_ref, o_ref):
        srcn = lax.rsqrt(jnp.maximum(op_ref[0] + op_ref[1], 1.0))
        r = jnp.dot(x_ref[...], w_ref[...],
                    preferred_element_type=jnp.float32) * srcn
        o_ref[0] = r[:, :DH]
        o_ref[1] = r[:, DH:]

    return pl.pallas_call(
        body,
        grid=(NPAD // RB,),
        in_specs=[
            pl.BlockSpec((RB, d), lambda i: (i, 0)),
            pl.BlockSpec((d, d), lambda i: (0, 0)),
            pl.BlockSpec((NC, RB, 1), lambda i: (0, i, 0)),
        ],
        out_specs=pl.BlockSpec((NC, RB, DH), lambda i: (0, i, 0)),
        out_shape=jax.ShapeDtypeStruct((NC, n, DH), jnp.float32),
    )(x, w, outp)


def _mid_call(parts, inp, outp, x, wb0, b0r, bb0r, w1, NPAD, RB):
    n, d = x.shape

    def body(p_ref, ip_ref, op_ref, x_ref, wb_ref, b0_ref,
             bb_ref, w1_ref, h1_ref, f1_ref):
        in_c = jnp.maximum(ip_ref[0] + ip_ref[1], 1.0)
        dstn = lax.rsqrt(in_c)
        ninv = 1.0 / in_c
        srcn = lax.rsqrt(jnp.maximum(op_ref[0] + op_ref[1], 1.0))
        agg = jnp.concatenate([p_ref[0], p_ref[1]], axis=-1)
        conv = agg * dstn + b0_ref[...]
        buf = jnp.dot(x_ref[...], wb_ref[...],
                      preferred_element_type=jnp.float32) + bb_ref[...]
        h1 = jnp.maximum(conv + ninv * buf, 0.0)
        h1_ref[...] = h1
        f1 = jnp.dot(h1, w1_ref[...],
                     preferred_element_type=jnp.float32) * srcn
        f1_ref[0] = f1[:, :DH]
        f1_ref[1] = f1[:, DH:]

    col = pl.BlockSpec((NC, RB, 1), lambda i: (0, i, 0))
    mat = pl.BlockSpec((RB, d), lambda i: (i, 0))
    wsp = pl.BlockSpec((d, d), lambda i: (0, 0))
    bsp = pl.BlockSpec((1, d), lambda i: (0, 0))
    psp = pl.BlockSpec((NC, RB, DH), lambda i: (0, i, 0))
    return pl.pallas_call(
        body,
        grid=(NPAD // RB,),
        in_specs=[psp, col, col, mat, wsp, bsp, bsp, wsp],
        out_specs=[mat, psp],
        out_shape=[
            jax.ShapeDtypeStruct((n, d), jnp.float32),
            jax.ShapeDtypeStruct((NC, n, DH), jnp.float32),
        ],
    )(parts, inp, outp, x, wb0, b0r, bb0r, w1)


def _final_call(parts, inp, x, h1, wb1a, wb1b, b1r, bb1r, NPAD, RB):
    n, d = x.shape

    def body(p_ref, ip_ref, x_ref, h1_ref, wa_ref, wb_ref,
             b1_ref, bb_ref, o_ref):
        in_c = jnp.maximum(ip_ref[0] + ip_ref[1], 1.0)
        dstn = lax.rsqrt(in_c)
        ninv = 1.0 / in_c
        agg = jnp.concatenate([p_ref[0], p_ref[1]], axis=-1)
        conv = agg * dstn + b1_ref[...]
        buf = (jnp.dot(x_ref[...], wa_ref[...],
                       preferred_element_type=jnp.float32)
               + jnp.dot(h1_ref[...], wb_ref[...],
                         preferred_element_type=jnp.float32) + bb_ref[...])
        o_ref[...] = conv + ninv * buf

    col = pl.BlockSpec((NC, RB, 1), lambda i: (0, i, 0))
    mat = pl.BlockSpec((RB, d), lambda i: (i, 0))
    wsp = pl.BlockSpec((d, d), lambda i: (0, 0))
    bsp = pl.BlockSpec((1, d), lambda i: (0, 0))
    psp = pl.BlockSpec((NC, RB, DH), lambda i: (0, i, 0))
    return pl.pallas_call(
        body,
        grid=(NPAD // RB,),
        in_specs=[psp, col, mat, mat, wsp, wsp, bsp, bsp],
        out_specs=mat,
        out_shape=jax.ShapeDtypeStruct((n, d), jnp.float32),
    )(parts, inp, x, h1, wb1a, wb1b, b1r, bb1r)


def kernel(features, edge_index, W0, b0, W1, b1, Wb0, bb0, Wb1, bb1):
    N, D = features.shape
    E = edge_index.shape[1]
    NPAD = N + PADR
    src = edge_index[0]
    dst = edge_index[1]

    CPW = -(-E // (NS * K))
    CPW = -(-CPW // NBUF) * NBUF
    E2 = NS * CPW * K
    pe = jnp.arange(E2 - E, dtype=jnp.int32)
    src_p = jnp.concatenate([src, pe % N]).reshape(NS, CPW, K)
    dst_p = jnp.concatenate([dst, N + pe % PADR]).reshape(NS, CPW, K)

    T = 2 * NPAD
    DE = 2 * E
    CD = -(-DE // (NW * K))
    pd = jnp.arange(NW * CD * K - DE, dtype=jnp.int32)
    degidx = jnp.concatenate([dst, src + NPAD, N + pd % PADR])
    degidx = degidx.reshape(NW, CD, K)

    zeros_e = jnp.zeros((T // NS,), jnp.float32)
    zeros_r = jnp.zeros((K, DH), jnp.float32)

    degparts = _make_deg_kernel(T, CD)(degidx, zeros_e)
    dp = degparts.reshape(NC, 2, NPAD, 1)
    inp = dp[:, 0]
    outp = dp[:, 1]

    RB = 1024
    b0r, bb0r = b0.reshape(1, D), bb0.reshape(1, D)
    b1r, bb1r = b1.reshape(1, D), bb1.reshape(1, D)
    wb1a, wb1b = Wb1[:D], Wb1[D:]

    pass_fn = _make_pass_kernel(NPAD, CPW)

    feat0 = _mm_scale_call(features, W0, outp, NPAD, RB)
    parts0 = pass_fn(src_p, dst_p, feat0, zeros_r)
    h1, feat1 = _mid_call(parts0, inp, outp,
                          features, Wb0, b0r, bb0r, W1, NPAD, RB)

    parts1 = pass_fn(src_p, dst_p, feat1, zeros_r)
    out = _final_call(parts1, inp,
                      features, h1, wb1a, wb1b, b1r, bb1r, NPAD, RB)
    return out


# speedup vs baseline: 10.9863x; 1.0105x over previous
import functools

import jax
import jax.numpy as jnp
from jax import lax
from jax.experimental import pallas as pl
from jax.experimental.pallas import tpu as pltpu
from jax.experimental.pallas import tpu_sc as plsc

NC = 2
NS = 16
NW = NC * NS
K = 128
PADR = 240
NBUF = 5
DH = 64


def _make_deg_kernel(T, CD):
    mesh = plsc.VectorSubcoreMesh(core_axis_name="c", subcore_axis_name="s")
    ept = T // NS

    @functools.partial(
        pl.kernel,
        mesh=mesh,
        out_type=jax.ShapeDtypeStruct((NC, T), jnp.float32),
        scratch_types=[
            pltpu.VMEM((CD, K), jnp.int32),
            pltpu.VMEM((K,), jnp.float32),
            pltpu.VMEM((ept,), jnp.float32),
            pltpu.VMEM_SHARED((T,), jnp.float32),
            pltpu.SemaphoreType.DMA,
        ],
    )
    def body(idx_hbm, zeros_hbm, out_hbm, idx_v, ones_v, zb_v, deg_sh, sem):
        cid = lax.axis_index("c")
        sid = lax.axis_index("s")
        wid = cid * NS + sid
        base = sid * ept
        for u in range(K // 16):
            ones_v[pl.ds(u * 16, 16)] = jnp.ones((16,), jnp.float32)
        pltpu.sync_copy(zeros_hbm, zb_v)
        pltpu.sync_copy(zb_v, deg_sh.at[pl.ds(base, ept)])
        pltpu.sync_copy(idx_hbm.at[wid], idx_v)
        plsc.subcore_barrier()

        def chunk(j, carry):
            pltpu.sync_copy(ones_v, deg_sh.at[idx_v.at[j]], add=True)
            return carry

        lax.fori_loop(0, CD, chunk, 0)
        plsc.subcore_barrier()
        pltpu.sync_copy(deg_sh.at[pl.ds(base, ept)],
                        out_hbm.at[cid, pl.ds(base, ept)])

    return body


def _make_pass_kernel(NPAD, CPW):
    mesh = plsc.VectorSubcoreMesh(core_axis_name="c", subcore_axis_name="s")
    rpt = NPAD // NS
    NG = CPW // NBUF

    @functools.partial(
        pl.kernel,
        mesh=mesh,
        out_type=jax.ShapeDtypeStruct((NC, NPAD, DH), jnp.float32),
        compiler_params=pltpu.CompilerParams(use_tc_tiling_on_sc=False),
        scratch_types=[
            pltpu.VMEM((CPW, K), jnp.int32),
            pltpu.VMEM((CPW, K), jnp.int32),
            pltpu.VMEM((NBUF, K, DH), jnp.float32),
            pltpu.VMEM_SHARED((NPAD, DH), jnp.float32),
            [pltpu.SemaphoreType.DMA] * NBUF,
            [pltpu.SemaphoreType.DMA] * NBUF,
        ],
    )
    def body(src_hbm, dst_hbm, feat_hbm, zeros_hbm, out_hbm,
             src_v, dst_v, rows_v, agg_sh, gsems, ssems):
        cid = lax.axis_index("c")
        sid = lax.axis_index("s")
        base = sid * rpt
        myfeat = feat_hbm.at[cid]
        pltpu.sync_copy(zeros_hbm, rows_v.at[0])
        for z in range(rpt // K):
            pltpu.sync_copy(rows_v.at[0], agg_sh.at[pl.ds(base + z * K, K)])
        pltpu.sync_copy(src_hbm.at[sid], src_v)
        pltpu.sync_copy(dst_hbm.at[sid], dst_v)
        plsc.subcore_barrier()

        for b in range(NBUF):
            pltpu.async_copy(myfeat.at[src_v.at[b]], rows_v.at[b], gsems[b])

        def group(g, carry):
            for b in range(NBUF):
                j = g * NBUF + b
                pltpu.make_async_copy(myfeat.at[src_v.at[j]],
                                      rows_v.at[b], gsems[b]).wait()
                pltpu.async_copy(rows_v.at[b], agg_sh.at[dst_v.at[j]],
                                 ssems[b], add=True)
            for b in range(NBUF):
                j2 = (g + 1) * NBUF + b
                pltpu.make_async_copy(rows_v.at[b], agg_sh.at[dst_v.at[0]],
                                      ssems[b]).wait()

                @pl.when(j2 < CPW)
                def _():
                    pltpu.async_copy(myfeat.at[src_v.at[j2]],
                                     rows_v.at[b], gsems[b])

            return carry

        lax.fori_loop(0, NG, group, 0)
        plsc.subcore_barrier()
        pltpu.sync_copy(agg_sh.at[pl.ds(base, rpt)],
                        out_hbm.at[cid, pl.ds(base, rpt)])

    return body


def _mm_scale_call(x, w, outp, NPAD, RB):
    n, d = x.shape

    def body(x_ref, w_ref, op_ref, o_ref):
        srcn = lax.rsqrt(jnp.maximum(op_ref[0] + op_ref[1], 1.0))
        r = jnp.dot(x_ref[...], w_ref[...],
                    preferred_element_type=jnp.float32) * srcn
        o_ref[0] = r[:, :DH]
        o_ref[1] = r[:, DH:]

    return pl.pallas_call(
        body,
        grid=(NPAD // RB,),
        in_specs=[
            pl.BlockSpec((RB, d), lambda i: (i, 0)),
            pl.BlockSpec((d, d), lambda i: (0, 0)),
            pl.BlockSpec((NC, RB, 1), lambda i: (0, i, 0)),
        ],
        out_specs=pl.BlockSpec((NC, RB, DH), lambda i: (0, i, 0)),
        out_shape=jax.ShapeDtypeStruct((NC, n, DH), jnp.float32),
    )(x, w, outp)


def _mid_call(parts, inp, outp, x, wb0, b0r, bb0r, w1, NPAD, RB):
    n, d = x.shape

    def body(p_ref, ip_ref, op_ref, x_ref, wb_ref, b0_ref,
             bb_ref, w1_ref, h1_ref, f1_ref):
        in_c = jnp.maximum(ip_ref[0] + ip_ref[1], 1.0)
        dstn = lax.rsqrt(in_c)
        ninv = 1.0 / in_c
        srcn = lax.rsqrt(jnp.maximum(op_ref[0] + op_ref[1], 1.0))
        agg = jnp.concatenate([p_ref[0], p_ref[1]], axis=-1)
        conv = agg * dstn + b0_ref[...]
        buf = jnp.dot(x_ref[...], wb_ref[...],
                      preferred_element_type=jnp.float32) + bb_ref[...]
        h1 = jnp.maximum(conv + ninv * buf, 0.0)
        h1_ref[...] = h1
        f1 = jnp.dot(h1, w1_ref[...],
                     preferred_element_type=jnp.float32) * srcn
        f1_ref[0] = f1[:, :DH]
        f1_ref[1] = f1[:, DH:]

    col = pl.BlockSpec((NC, RB, 1), lambda i: (0, i, 0))
    mat = pl.BlockSpec((RB, d), lambda i: (i, 0))
    wsp = pl.BlockSpec((d, d), lambda i: (0, 0))
    bsp = pl.BlockSpec((1, d), lambda i: (0, 0))
    psp = pl.BlockSpec((NC, RB, DH), lambda i: (0, i, 0))
    return pl.pallas_call(
        body,
        grid=(NPAD // RB,),
        in_specs=[psp, col, col, mat, wsp, bsp, bsp, wsp],
        out_specs=[mat, psp],
        out_shape=[
            jax.ShapeDtypeStruct((n, d), jnp.float32),
            jax.ShapeDtypeStruct((NC, n, DH), jnp.float32),
        ],
    )(parts, inp, outp, x, wb0, b0r, bb0r, w1)


def _final_call(parts, inp, x, h1, wb1a, wb1b, b1r, bb1r, NPAD, RB):
    n, d = x.shape

    def body(p_ref, ip_ref, x_ref, h1_ref, wa_ref, wb_ref,
             b1_ref, bb_ref, o_ref):
        in_c = jnp.maximum(ip_ref[0] + ip_ref[1], 1.0)
        dstn = lax.rsqrt(in_c)
        ninv = 1.0 / in_c
        agg = jnp.concatenate([p_ref[0], p_ref[1]], axis=-1)
        conv = agg * dstn + b1_ref[...]
        buf = (jnp.dot(x_ref[...], wa_ref[...],
                       preferred_element_type=jnp.float32)
               + jnp.dot(h1_ref[...], wb_ref[...],
                         preferred_element_type=jnp.float32) + bb_ref[...])
        o_ref[...] = conv + ninv * buf

    col = pl.BlockSpec((NC, RB, 1), lambda i: (0, i, 0))
    mat = pl.BlockSpec((RB, d), lambda i: (i, 0))
    wsp = pl.BlockSpec((d, d), lambda i: (0, 0))
    bsp = pl.BlockSpec((1, d), lambda i: (0, 0))
    psp = pl.BlockSpec((NC, RB, DH), lambda i: (0, i, 0))
    return pl.pallas_call(
        body,
        grid=(NPAD // RB,),
        in_specs=[psp, col, mat, mat, wsp, wsp, bsp, bsp],
        out_specs=mat,
        out_shape=jax.ShapeDtypeStruct((n, d), jnp.float32),
    )(parts, inp, x, h1, wb1a, wb1b, b1r, bb1r)


def kernel(features, edge_index, W0, b0, W1, b1, Wb0, bb0, Wb1, bb1):
    N, D = features.shape
    E = edge_index.shape[1]
    NPAD = N + PADR
    src = edge_index[0]
    dst = edge_index[1]

    CPW = -(-E // (NS * K))
    CPW = -(-CPW // NBUF) * NBUF
    E2 = NS * CPW * K
    pe = jnp.arange(E2 - E, dtype=jnp.int32)
    src_p = jnp.concatenate([src, pe % N]).reshape(NS, CPW, K)
    dst_p = jnp.concatenate([dst, N + pe % PADR]).reshape(NS, CPW, K)

    T = 2 * NPAD
    DE = 2 * E
    CD = -(-DE // (NW * K))
    pd = jnp.arange(NW * CD * K - DE, dtype=jnp.int32)
    degidx = jnp.concatenate([dst, src + NPAD, N + pd % PADR])
    degidx = degidx.reshape(NW, CD, K)

    zeros_e = jnp.zeros((T // NS,), jnp.float32)
    zeros_r = jnp.zeros((K, DH), jnp.float32)

    degparts = _make_deg_kernel(T, CD)(degidx, zeros_e)
    dp = degparts.reshape(NC, 2, NPAD, 1)
    inp = dp[:, 0]
    outp = dp[:, 1]

    RB = 1024
    b0r, bb0r = b0.reshape(1, D), bb0.reshape(1, D)
    b1r, bb1r = b1.reshape(1, D), bb1.reshape(1, D)
    wb1a, wb1b = Wb1[:D], Wb1[D:]

    pass_fn = _make_pass_kernel(NPAD, CPW)

    feat0 = _mm_scale_call(features, W0, outp, NPAD, RB)
    parts0 = pass_fn(src_p, dst_p, feat0, zeros_r)
    h1, feat1 = _mid_call(parts0, inp, outp,
                          features, Wb0, b0r, bb0r, W1, NPAD, RB)

    parts1 = pass_fn(src_p, dst_p, feat1, zeros_r)
    out = _final_call(parts1, inp,
                      features, h1, wb1a, wb1b, b1r, bb1r, NPAD, RB)
    return out


# speedup vs baseline: 11.2962x; 1.0282x over previous
import functools

import jax
import jax.numpy as jnp
from jax import lax
from jax.experimental import pallas as pl
from jax.experimental.pallas import tpu as pltpu
from jax.experimental.pallas import tpu_sc as plsc

NC = 2
NS = 16
NW = NC * NS
K = 128
PADR = 240
NBUF = 5
DH = 64


def _make_deg_kernel(T, CD):
    mesh = plsc.VectorSubcoreMesh(core_axis_name="c", subcore_axis_name="s")
    ept = T // NS

    NB = 8

    @functools.partial(
        pl.kernel,
        mesh=mesh,
        out_type=jax.ShapeDtypeStruct((NC, T), jnp.float32),
        scratch_types=[
            pltpu.VMEM((CD, K), jnp.int32),
            pltpu.VMEM((K,), jnp.float32),
            pltpu.VMEM((ept,), jnp.float32),
            pltpu.VMEM_SHARED((T,), jnp.float32),
            [pltpu.SemaphoreType.DMA] * NB,
        ],
    )
    def body(idx_hbm, zeros_hbm, out_hbm, idx_v, ones_v, zb_v, deg_sh, sems):
        cid = lax.axis_index("c")
        sid = lax.axis_index("s")
        wid = cid * NS + sid
        base = sid * ept
        for u in range(K // 16):
            ones_v[pl.ds(u * 16, 16)] = jnp.ones((16,), jnp.float32)
        pltpu.sync_copy(zeros_hbm, zb_v)
        pltpu.sync_copy(zb_v, deg_sh.at[pl.ds(base, ept)])
        pltpu.sync_copy(idx_hbm.at[wid], idx_v)
        plsc.subcore_barrier()

        def group(g, carry):
            for b in range(NB):
                pltpu.async_copy(ones_v, deg_sh.at[idx_v.at[g * NB + b]],
                                 sems[b], add=True)
            for b in range(NB):
                pltpu.make_async_copy(ones_v, deg_sh.at[idx_v.at[0]],
                                      sems[b]).wait()
            return carry

        lax.fori_loop(0, CD // NB, group, 0)
        plsc.subcore_barrier()
        pltpu.sync_copy(deg_sh.at[pl.ds(base, ept)],
                        out_hbm.at[cid, pl.ds(base, ept)])

    return body


def _make_pass_kernel(NPAD, CPW):
    mesh = plsc.VectorSubcoreMesh(core_axis_name="c", subcore_axis_name="s")
    rpt = NPAD // NS
    NG = CPW // NBUF

    @functools.partial(
        pl.kernel,
        mesh=mesh,
        out_type=jax.ShapeDtypeStruct((NC, NPAD, DH), jnp.float32),
        compiler_params=pltpu.CompilerParams(use_tc_tiling_on_sc=False),
        scratch_types=[
            pltpu.VMEM((CPW, K), jnp.int32),
            pltpu.VMEM((CPW, K), jnp.int32),
            pltpu.VMEM((NBUF, K, DH), jnp.float32),
            pltpu.VMEM_SHARED((NPAD, DH), jnp.float32),
            [pltpu.SemaphoreType.DMA] * NBUF,
            [pltpu.SemaphoreType.DMA] * NBUF,
        ],
    )
    def body(src_hbm, dst_hbm, feat_hbm, zeros_hbm, out_hbm,
             src_v, dst_v, rows_v, agg_sh, gsems, ssems):
        cid = lax.axis_index("c")
        sid = lax.axis_index("s")
        base = sid * rpt
        myfeat = feat_hbm.at[cid]
        pltpu.sync_copy(zeros_hbm, rows_v.at[0])
        for z in range(rpt // K):
            pltpu.sync_copy(rows_v.at[0], agg_sh.at[pl.ds(base + z * K, K)])
        pltpu.sync_copy(src_hbm.at[sid], src_v)
        pltpu.sync_copy(dst_hbm.at[sid], dst_v)
        plsc.subcore_barrier()

        for b in range(NBUF):
            pltpu.async_copy(myfeat.at[src_v.at[b]], rows_v.at[b], gsems[b])

        def group(g, carry):
            for b in range(NBUF):
                j = g * NBUF + b
                pltpu.make_async_copy(myfeat.at[src_v.at[j]],
                                      rows_v.at[b], gsems[b]).wait()
                pltpu.async_copy(rows_v.at[b], agg_sh.at[dst_v.at[j]],
                                 ssems[b], add=True)
            for b in range(NBUF):
                j2 = (g + 1) * NBUF + b
                pltpu.make_async_copy(rows_v.at[b], agg_sh.at[dst_v.at[0]],
                                      ssems[b]).wait()

                @pl.when(j2 < CPW)
                def _():
                    pltpu.async_copy(myfeat.at[src_v.at[j2]],
                                     rows_v.at[b], gsems[b])

            return carry

        lax.fori_loop(0, NG, group, 0)
        plsc.subcore_barrier()
        pltpu.sync_copy(agg_sh.at[pl.ds(base, rpt)],
                        out_hbm.at[cid, pl.ds(base, rpt)])

    return body


def _mm_scale_call(x, w, outp, NPAD, RB):
    n, d = x.shape

    def body(x_ref, w_ref, op_ref, o_ref):
        srcn = lax.rsqrt(jnp.maximum(op_ref[0] + op_ref[1], 1.0))
        r = jnp.dot(x_ref[...], w_ref[...],
                    preferred_element_type=jnp.float32) * srcn
        o_ref[0] = r[:, :DH]
        o_ref[1] = r[:, DH:]

    return pl.pallas_call(
        body,
        grid=(NPAD // RB,),
        in_specs=[
            pl.BlockSpec((RB, d), lambda i: (i, 0)),
            pl.BlockSpec((d, d), lambda i: (0, 0)),
            pl.BlockSpec((NC, RB, 1), lambda i: (0, i, 0)),
        ],
        out_specs=pl.BlockSpec((NC, RB, DH), lambda i: (0, i, 0)),
        out_shape=jax.ShapeDtypeStruct((NC, n, DH), jnp.float32),
    )(x, w, outp)


def _mid_call(parts, inp, outp, x, wb0, b0r, bb0r, w1, NPAD, RB):
    n, d = x.shape

    def body(p_ref, ip_ref, op_ref, x_ref, wb_ref, b0_ref,
             bb_ref, w1_ref, h1_ref, f1_ref):
        in_c = jnp.maximum(ip_ref[0] + ip_ref[1], 1.0)
        dstn = lax.rsqrt(in_c)
        ninv = 1.0 / in_c
        srcn = lax.rsqrt(jnp.maximum(op_ref[0] + op_ref[1], 1.0))
        agg = jnp.concatenate([p_ref[0], p_ref[1]], axis=-1)
        conv = agg * dstn + b0_ref[...]
        buf = jnp.dot(x_ref[...], wb_ref[...],
                      preferred_element_type=jnp.float32) + bb_ref[...]
        h1 = jnp.maximum(conv + ninv * buf, 0.0)
        h1_ref[...] = h1
        f1 = jnp.dot(h1, w1_ref[...],
                     preferred_element_type=jnp.float32) * srcn
        f1_ref[0] = f1[:, :DH]
        f1_ref[1] = f1[:, DH:]

    col = pl.BlockSpec((NC, RB, 1), lambda i: (0, i, 0))
    mat = pl.BlockSpec((RB, d), lambda i: (i, 0))
    wsp = pl.BlockSpec((d, d), lambda i: (0, 0))
    bsp = pl.BlockSpec((1, d), lambda i: (0, 0))
    psp = pl.BlockSpec((NC, RB, DH), lambda i: (0, i, 0))
    return pl.pallas_call(
        body,
        grid=(NPAD // RB,),
        in_specs=[psp, col, col, mat, wsp, bsp, bsp, wsp],
        out_specs=[mat, psp],
        out_shape=[
            jax.ShapeDtypeStruct((n, d), jnp.float32),
            jax.ShapeDtypeStruct((NC, n, DH), jnp.float32),
        ],
    )(parts, inp, outp, x, wb0, b0r, bb0r, w1)


def _final_call(parts, inp, x, h1, wb1a, wb1b, b1r, bb1r, NPAD, RB):
    n, d = x.shape

    def body(p_ref, ip_ref, x_ref, h1_ref, wa_ref, wb_ref,
             b1_ref, bb_ref, o_ref):
        in_c = jnp.maximum(ip_ref[0] + ip_ref[1], 1.0)
        dstn = lax.rsqrt(in_c)
        ninv = 1.0 / in_c
        agg = jnp.concatenate([p_ref[0], p_ref[1]], axis=-1)
        conv = agg * dstn + b1_ref[...]
        buf = (jnp.dot(x_ref[...], wa_ref[...],
                       preferred_element_type=jnp.float32)
               + jnp.dot(h1_ref[...], wb_ref[...],
                         preferred_element_type=jnp.float32) + bb_ref[...])
        o_ref[...] = conv + ninv * buf

    col = pl.BlockSpec((NC, RB, 1), lambda i: (0, i, 0))
    mat = pl.BlockSpec((RB, d), lambda i: (i, 0))
    wsp = pl.BlockSpec((d, d), lambda i: (0, 0))
    bsp = pl.BlockSpec((1, d), lambda i: (0, 0))
    psp = pl.BlockSpec((NC, RB, DH), lambda i: (0, i, 0))
    return pl.pallas_call(
        body,
        grid=(NPAD // RB,),
        in_specs=[psp, col, mat, mat, wsp, wsp, bsp, bsp],
        out_specs=mat,
        out_shape=jax.ShapeDtypeStruct((n, d), jnp.float32),
    )(parts, inp, x, h1, wb1a, wb1b, b1r, bb1r)


def kernel(features, edge_index, W0, b0, W1, b1, Wb0, bb0, Wb1, bb1):
    N, D = features.shape
    E = edge_index.shape[1]
    NPAD = N + PADR
    src = edge_index[0]
    dst = edge_index[1]

    CPW = -(-E // (NS * K))
    CPW = -(-CPW // NBUF) * NBUF
    E2 = NS * CPW * K
    pe = jnp.arange(E2 - E, dtype=jnp.int32)
    src_p = jnp.concatenate([src, pe % N]).reshape(NS, CPW, K)
    dst_p = jnp.concatenate([dst, N + pe % PADR]).reshape(NS, CPW, K)

    T = 2 * NPAD
    DE = 2 * E
    CD = -(-DE // (NW * K))
    CD = -(-CD // 8) * 8
    pd = jnp.arange(NW * CD * K - DE, dtype=jnp.int32)
    degidx = jnp.concatenate([dst, src + NPAD, N + pd % PADR])
    degidx = degidx.reshape(NW, CD, K)

    zeros_e = jnp.zeros((T // NS,), jnp.float32)
    zeros_r = jnp.zeros((K, DH), jnp.float32)

    degparts = _make_deg_kernel(T, CD)(degidx, zeros_e)
    dp = degparts.reshape(NC, 2, NPAD, 1)
    inp = dp[:, 0]
    outp = dp[:, 1]

    RB = 1024
    b0r, bb0r = b0.reshape(1, D), bb0.reshape(1, D)
    b1r, bb1r = b1.reshape(1, D), bb1.reshape(1, D)
    wb1a, wb1b = Wb1[:D], Wb1[D:]

    pass_fn = _make_pass_kernel(NPAD, CPW)

    feat0 = _mm_scale_call(features, W0, outp, NPAD, RB)
    parts0 = pass_fn(src_p, dst_p, feat0, zeros_r)
    h1, feat1 = _mid_call(parts0, inp, outp,
                          features, Wb0, b0r, bb0r, W1, NPAD, RB)

    parts1 = pass_fn(src_p, dst_p, feat1, zeros_r)
    out = _final_call(parts1, inp,
                      features, h1, wb1a, wb1b, b1r, bb1r, NPAD, RB)
    return out


# speedup vs baseline: 11.3027x; 1.0006x over previous
import functools

import jax
import jax.numpy as jnp
from jax import lax
from jax.experimental import pallas as pl
from jax.experimental.pallas import tpu as pltpu
from jax.experimental.pallas import tpu_sc as plsc

NC = 2
NS = 16
NW = NC * NS
K = 128
PADR = 240
NBUF = 5
DH = 64


def _make_deg_kernel(T, CD):
    mesh = plsc.VectorSubcoreMesh(core_axis_name="c", subcore_axis_name="s")
    ept = T // NS

    NB = 8

    @functools.partial(
        pl.kernel,
        mesh=mesh,
        out_type=jax.ShapeDtypeStruct((NC, T), jnp.float32),
        compiler_params=pltpu.CompilerParams(skip_device_barrier=True),
        scratch_types=[
            pltpu.VMEM((CD, K), jnp.int32),
            pltpu.VMEM((K,), jnp.float32),
            pltpu.VMEM((ept,), jnp.float32),
            pltpu.VMEM_SHARED((T,), jnp.float32),
            [pltpu.SemaphoreType.DMA] * NB,
        ],
    )
    def body(idx_hbm, zeros_hbm, out_hbm, idx_v, ones_v, zb_v, deg_sh, sems):
        cid = lax.axis_index("c")
        sid = lax.axis_index("s")
        wid = cid * NS + sid
        base = sid * ept
        for u in range(K // 16):
            ones_v[pl.ds(u * 16, 16)] = jnp.ones((16,), jnp.float32)
        pltpu.sync_copy(zeros_hbm, zb_v)
        pltpu.sync_copy(zb_v, deg_sh.at[pl.ds(base, ept)])
        pltpu.sync_copy(idx_hbm.at[wid], idx_v)
        plsc.subcore_barrier()

        def group(g, carry):
            for b in range(NB):
                pltpu.async_copy(ones_v, deg_sh.at[idx_v.at[g * NB + b]],
                                 sems[b], add=True)
            for b in range(NB):
                pltpu.make_async_copy(ones_v, deg_sh.at[idx_v.at[0]],
                                      sems[b]).wait()
            return carry

        lax.fori_loop(0, CD // NB, group, 0)
        plsc.subcore_barrier()
        pltpu.sync_copy(deg_sh.at[pl.ds(base, ept)],
                        out_hbm.at[cid, pl.ds(base, ept)])

    return body


def _make_pass_kernel(NPAD, CPW):
    mesh = plsc.VectorSubcoreMesh(core_axis_name="c", subcore_axis_name="s")
    rpt = NPAD // NS
    NG = CPW // NBUF

    @functools.partial(
        pl.kernel,
        mesh=mesh,
        out_type=jax.ShapeDtypeStruct((NC, NPAD, DH), jnp.float32),
        compiler_params=pltpu.CompilerParams(use_tc_tiling_on_sc=False,
                                             skip_device_barrier=True),
        scratch_types=[
            pltpu.VMEM((CPW, K), jnp.int32),
            pltpu.VMEM((CPW, K), jnp.int32),
            pltpu.VMEM((NBUF, K, DH), jnp.float32),
            pltpu.VMEM_SHARED((NPAD, DH), jnp.float32),
            [pltpu.SemaphoreType.DMA] * NBUF,
            [pltpu.SemaphoreType.DMA] * NBUF,
        ],
    )
    def body(src_hbm, dst_hbm, feat_hbm, zeros_hbm, out_hbm,
             src_v, dst_v, rows_v, agg_sh, gsems, ssems):
        cid = lax.axis_index("c")
        sid = lax.axis_index("s")
        base = sid * rpt
        myfeat = feat_hbm.at[cid]
        pltpu.sync_copy(zeros_hbm, rows_v.at[0])
        for z in range(rpt // K):
            pltpu.sync_copy(rows_v.at[0], agg_sh.at[pl.ds(base + z * K, K)])
        pltpu.sync_copy(src_hbm.at[sid], src_v)
        pltpu.sync_copy(dst_hbm.at[sid], dst_v)
        plsc.subcore_barrier()

        for b in range(NBUF):
            pltpu.async_copy(myfeat.at[src_v.at[b]], rows_v.at[b], gsems[b])

        def group(g, carry):
            for b in range(NBUF):
                j = g * NBUF + b
                pltpu.make_async_copy(myfeat.at[src_v.at[j]],
                                      rows_v.at[b], gsems[b]).wait()
                pltpu.async_copy(rows_v.at[b], agg_sh.at[dst_v.at[j]],
                                 ssems[b], add=True)
            for b in range(NBUF):
                j2 = (g + 1) * NBUF + b
                pltpu.make_async_copy(rows_v.at[b], agg_sh.at[dst_v.at[0]],
                                      ssems[b]).wait()

                @pl.when(j2 < CPW)
                def _():
                    pltpu.async_copy(myfeat.at[src_v.at[j2]],
                                     rows_v.at[b], gsems[b])

            return carry

        lax.fori_loop(0, NG, group, 0)
        plsc.subcore_barrier()
        pltpu.sync_copy(agg_sh.at[pl.ds(base, rpt)],
                        out_hbm.at[cid, pl.ds(base, rpt)])

    return body


def _mm_scale_call(x, w, outp, NPAD, RB):
    n, d = x.shape

    def body(x_ref, w_ref, op_ref, o_ref):
        srcn = lax.rsqrt(jnp.maximum(op_ref[0] + op_ref[1], 1.0))
        r = jnp.dot(x_ref[...], w_ref[...],
                    preferred_element_type=jnp.float32) * srcn
        o_ref[0] = r[:, :DH]
        o_ref[1] = r[:, DH:]

    return pl.pallas_call(
        body,
        grid=(NPAD // RB,),
        in_specs=[
            pl.BlockSpec((RB, d), lambda i: (i, 0)),
            pl.BlockSpec((d, d), lambda i: (0, 0)),
            pl.BlockSpec((NC, RB, 1), lambda i: (0, i, 0)),
        ],
        out_specs=pl.BlockSpec((NC, RB, DH), lambda i: (0, i, 0)),
        out_shape=jax.ShapeDtypeStruct((NC, n, DH), jnp.float32),
    )(x, w, outp)


def _mid_call(parts, inp, outp, x, wb0, b0r, bb0r, w1, NPAD, RB):
    n, d = x.shape

    def body(p_ref, ip_ref, op_ref, x_ref, wb_ref, b0_ref,
             bb_ref, w1_ref, h1_ref, f1_ref):
        in_c = jnp.maximum(ip_ref[0] + ip_ref[1], 1.0)
        dstn = lax.rsqrt(in_c)
        ninv = 1.0 / in_c
        srcn = lax.rsqrt(jnp.maximum(op_ref[0] + op_ref[1], 1.0))
        agg = jnp.concatenate([p_ref[0], p_ref[1]], axis=-1)
        conv = agg * dstn + b0_ref[...]
        buf = jnp.dot(x_ref[...], wb_ref[...],
                      preferred_element_type=jnp.float32) + bb_ref[...]
        h1 = jnp.maximum(conv + ninv * buf, 0.0)
        h1_ref[...] = h1
        f1 = jnp.dot(h1, w1_ref[...],
                     preferred_element_type=jnp.float32) * srcn
        f1_ref[0] = f1[:, :DH]
        f1_ref[1] = f1[:, DH:]

    col = pl.BlockSpec((NC, RB, 1), lambda i: (0, i, 0))
    mat = pl.BlockSpec((RB, d), lambda i: (i, 0))
    wsp = pl.BlockSpec((d, d), lambda i: (0, 0))
    bsp = pl.BlockSpec((1, d), lambda i: (0, 0))
    psp = pl.BlockSpec((NC, RB, DH), lambda i: (0, i, 0))
    return pl.pallas_call(
        body,
        grid=(NPAD // RB,),
        in_specs=[psp, col, col, mat, wsp, bsp, bsp, wsp],
        out_specs=[mat, psp],
        out_shape=[
            jax.ShapeDtypeStruct((n, d), jnp.float32),
            jax.ShapeDtypeStruct((NC, n, DH), jnp.float32),
        ],
    )(parts, inp, outp, x, wb0, b0r, bb0r, w1)


def _final_call(parts, inp, x, h1, wb1a, wb1b, b1r, bb1r, NPAD, RB):
    n, d = x.shape

    def body(p_ref, ip_ref, x_ref, h1_ref, wa_ref, wb_ref,
             b1_ref, bb_ref, o_ref):
        in_c = jnp.maximum(ip_ref[0] + ip_ref[1], 1.0)
        dstn = lax.rsqrt(in_c)
        ninv = 1.0 / in_c
        agg = jnp.concatenate([p_ref[0], p_ref[1]], axis=-1)
        conv = agg * dstn + b1_ref[...]
        buf = (jnp.dot(x_ref[...], wa_ref[...],
                       preferred_element_type=jnp.float32)
               + jnp.dot(h1_ref[...], wb_ref[...],
                         preferred_element_type=jnp.float32) + bb_ref[...])
        o_ref[...] = conv + ninv * buf

    col = pl.BlockSpec((NC, RB, 1), lambda i: (0, i, 0))
    mat = pl.BlockSpec((RB, d), lambda i: (i, 0))
    wsp = pl.BlockSpec((d, d), lambda i: (0, 0))
    bsp = pl.BlockSpec((1, d), lambda i: (0, 0))
    psp = pl.BlockSpec((NC, RB, DH), lambda i: (0, i, 0))
    return pl.pallas_call(
        body,
        grid=(NPAD // RB,),
        in_specs=[psp, col, mat, mat, wsp, wsp, bsp, bsp],
        out_specs=mat,
        out_shape=jax.ShapeDtypeStruct((n, d), jnp.float32),
    )(parts, inp, x, h1, wb1a, wb1b, b1r, bb1r)


def kernel(features, edge_index, W0, b0, W1, b1, Wb0, bb0, Wb1, bb1):
    N, D = features.shape
    E = edge_index.shape[1]
    NPAD = N + PADR
    src = edge_index[0]
    dst = edge_index[1]

    CPW = -(-E // (NS * K))
    CPW = -(-CPW // NBUF) * NBUF
    E2 = NS * CPW * K
    pe = jnp.arange(E2 - E, dtype=jnp.int32)
    src_p = jnp.concatenate([src, pe % N]).reshape(NS, CPW, K)
    dst_p = jnp.concatenate([dst, N + pe % PADR]).reshape(NS, CPW, K)

    T = 2 * NPAD
    DE = 2 * E
    CD = -(-DE // (NW * K))
    CD = -(-CD // 8) * 8
    pd = jnp.arange(NW * CD * K - DE, dtype=jnp.int32)
    degidx = jnp.concatenate([dst, src + NPAD, N + pd % PADR])
    degidx = degidx.reshape(NW, CD, K)

    zeros_e = jnp.zeros((T // NS,), jnp.float32)
    zeros_r = jnp.zeros((K, DH), jnp.float32)

    degparts = _make_deg_kernel(T, CD)(degidx, zeros_e)
    dp = degparts.reshape(NC, 2, NPAD, 1)
    inp = dp[:, 0]
    outp = dp[:, 1]

    RB = 1024
    b0r, bb0r = b0.reshape(1, D), bb0.reshape(1, D)
    b1r, bb1r = b1.reshape(1, D), bb1.reshape(1, D)
    wb1a, wb1b = Wb1[:D], Wb1[D:]

    pass_fn = _make_pass_kernel(NPAD, CPW)

    feat0 = _mm_scale_call(features, W0, outp, NPAD, RB)
    parts0 = pass_fn(src_p, dst_p, feat0, zeros_r)
    h1, feat1 = _mid_call(parts0, inp, outp,
                          features, Wb0, b0r, bb0r, W1, NPAD, RB)

    parts1 = pass_fn(src_p, dst_p, feat1, zeros_r)
    out = _final_call(parts1, inp,
                      features, h1, wb1a, wb1b, b1r, bb1r, NPAD, RB)
    return out


# speedup vs baseline: 11.3543x; 1.0046x over previous
import functools

import jax
import jax.numpy as jnp
from jax import lax
from jax.experimental import pallas as pl
from jax.experimental.pallas import tpu as pltpu
from jax.experimental.pallas import tpu_sc as plsc

NC = 2
NS = 16
NW = NC * NS
K = 128
PADR = 240
NBUF = 5
DH = 64


def _make_deg_kernel(T, CD):
    mesh = plsc.VectorSubcoreMesh(core_axis_name="c", subcore_axis_name="s")
    ept = T // NS

    NB = 8

    @functools.partial(
        pl.kernel,
        mesh=mesh,
        out_type=jax.ShapeDtypeStruct((NC, T), jnp.float32),
        scratch_types=[
            pltpu.VMEM((CD, K), jnp.int32),
            pltpu.VMEM((K,), jnp.float32),
            pltpu.VMEM((ept,), jnp.float32),
            pltpu.VMEM_SHARED((T,), jnp.float32),
            [pltpu.SemaphoreType.DMA] * NB,
        ],
    )
    def body(idx_hbm, zeros_hbm, out_hbm, idx_v, ones_v, zb_v, deg_sh, sems):
        cid = lax.axis_index("c")
        sid = lax.axis_index("s")
        wid = cid * NS + sid
        base = sid * ept
        for u in range(K // 16):
            ones_v[pl.ds(u * 16, 16)] = jnp.ones((16,), jnp.float32)
        pltpu.sync_copy(zeros_hbm, zb_v)
        pltpu.sync_copy(zb_v, deg_sh.at[pl.ds(base, ept)])
        pltpu.sync_copy(idx_hbm.at[wid], idx_v)
        plsc.subcore_barrier()

        def group(g, carry):
            for b in range(NB):
                pltpu.async_copy(ones_v, deg_sh.at[idx_v.at[g * NB + b]],
                                 sems[b], add=True)
            for b in range(NB):
                pltpu.make_async_copy(ones_v, deg_sh.at[idx_v.at[0]],
                                      sems[b]).wait()
            return carry

        lax.fori_loop(0, CD // NB, group, 0)
        plsc.subcore_barrier()
        pltpu.sync_copy(deg_sh.at[pl.ds(base, ept)],
                        out_hbm.at[cid, pl.ds(base, ept)])

    return body


def _make_pass_kernel(NPAD, CPW):
    mesh = plsc.VectorSubcoreMesh(core_axis_name="c", subcore_axis_name="s")
    rpt = NPAD // NS
    NG = CPW // NBUF

    @functools.partial(
        pl.kernel,
        mesh=mesh,
        out_type=jax.ShapeDtypeStruct((NC, NPAD, DH), jnp.float32),
        compiler_params=pltpu.CompilerParams(use_tc_tiling_on_sc=False),
        scratch_types=[
            pltpu.VMEM((CPW, K), jnp.int32),
            pltpu.VMEM((CPW, K), jnp.int32),
            pltpu.VMEM((NBUF, K, DH), jnp.float32),
            pltpu.VMEM_SHARED((NPAD, DH), jnp.float32),
            [pltpu.SemaphoreType.DMA] * NBUF,
            [pltpu.SemaphoreType.DMA] * NBUF,
        ],
    )
    def body(src_hbm, dst_hbm, feat_hbm, zeros_hbm, out_hbm,
             src_v, dst_v, rows_v, agg_sh, gsems, ssems):
        cid = lax.axis_index("c")
        sid = lax.axis_index("s")
        base = sid * rpt
        myfeat = feat_hbm.at[cid]
        pltpu.async_copy(src_hbm.at[sid], src_v, gsems[0])
        pltpu.async_copy(dst_hbm.at[sid], dst_v, gsems[1])
        pltpu.sync_copy(zeros_hbm, rows_v.at[0])
        for z in range(rpt // K):
            pltpu.sync_copy(rows_v.at[0], agg_sh.at[pl.ds(base + z * K, K)])
        pltpu.make_async_copy(src_hbm.at[sid], src_v, gsems[0]).wait()
        pltpu.make_async_copy(dst_hbm.at[sid], dst_v, gsems[1]).wait()
        plsc.subcore_barrier()

        for b in range(NBUF):
            pltpu.async_copy(myfeat.at[src_v.at[b]], rows_v.at[b], gsems[b])

        def group(g, carry):
            for b in range(NBUF):
                j = g * NBUF + b
                pltpu.make_async_copy(myfeat.at[src_v.at[j]],
                                      rows_v.at[b], gsems[b]).wait()
                pltpu.async_copy(rows_v.at[b], agg_sh.at[dst_v.at[j]],
                                 ssems[b], add=True)
            for b in range(NBUF):
                j2 = (g + 1) * NBUF + b
                pltpu.make_async_copy(rows_v.at[b], agg_sh.at[dst_v.at[0]],
                                      ssems[b]).wait()

                @pl.when(j2 < CPW)
                def _():
                    pltpu.async_copy(myfeat.at[src_v.at[j2]],
                                     rows_v.at[b], gsems[b])

            return carry

        lax.fori_loop(0, NG, group, 0)
        plsc.subcore_barrier()
        pltpu.sync_copy(agg_sh.at[pl.ds(base, rpt)],
                        out_hbm.at[cid, pl.ds(base, rpt)])

    return body


def _mm_scale_call(x, w, outp, NPAD, RB):
    n, d = x.shape

    def body(x_ref, w_ref, op_ref, o_ref):
        srcn = lax.rsqrt(jnp.maximum(op_ref[0] + op_ref[1], 1.0))
        r = jnp.dot(x_ref[...], w_ref[...],
                    preferred_element_type=jnp.float32) * srcn
        o_ref[0] = r[:, :DH]
        o_ref[1] = r[:, DH:]

    return pl.pallas_call(
        body,
        grid=(NPAD // RB,),
        in_specs=[
            pl.BlockSpec((RB, d), lambda i: (i, 0)),
            pl.BlockSpec((d, d), lambda i: (0, 0)),
            pl.BlockSpec((NC, RB, 1), lambda i: (0, i, 0)),
        ],
        out_specs=pl.BlockSpec((NC, RB, DH), lambda i: (0, i, 0)),
        out_shape=jax.ShapeDtypeStruct((NC, n, DH), jnp.float32),
    )(x, w, outp)


def _mid_call(parts, inp, outp, x, wb0, b0r, bb0r, w1, NPAD, RB):
    n, d = x.shape

    def body(p_ref, ip_ref, op_ref, x_ref, wb_ref, b0_ref,
             bb_ref, w1_ref, h1_ref, f1_ref):
        in_c = jnp.maximum(ip_ref[0] + ip_ref[1], 1.0)
        dstn = lax.rsqrt(in_c)
        ninv = 1.0 / in_c
        srcn = lax.rsqrt(jnp.maximum(op_ref[0] + op_ref[1], 1.0))
        agg = jnp.concatenate([p_ref[0], p_ref[1]], axis=-1)
        conv = agg * dstn + b0_ref[...]
        buf = jnp.dot(x_ref[...], wb_ref[...],
                      preferred_element_type=jnp.float32) + bb_ref[...]
        h1 = jnp.maximum(conv + ninv * buf, 0.0)
        h1_ref[...] = h1
        f1 = jnp.dot(h1, w1_ref[...],
                     preferred_element_type=jnp.float32) * srcn
        f1_ref[0] = f1[:, :DH]
        f1_ref[1] = f1[:, DH:]

    col = pl.BlockSpec((NC, RB, 1), lambda i: (0, i, 0))
    mat = pl.BlockSpec((RB, d), lambda i: (i, 0))
    wsp = pl.BlockSpec((d, d), lambda i: (0, 0))
    bsp = pl.BlockSpec((1, d), lambda i: (0, 0))
    psp = pl.BlockSpec((NC, RB, DH), lambda i: (0, i, 0))
    return pl.pallas_call(
        body,
        grid=(NPAD // RB,),
        in_specs=[psp, col, col, mat, wsp, bsp, bsp, wsp],
        out_specs=[mat, psp],
        out_shape=[
            jax.ShapeDtypeStruct((n, d), jnp.float32),
            jax.ShapeDtypeStruct((NC, n, DH), jnp.float32),
        ],
    )(parts, inp, outp, x, wb0, b0r, bb0r, w1)


def _final_call(parts, inp, x, h1, wb1a, wb1b, b1r, bb1r, NPAD, RB):
    n, d = x.shape

    def body(p_ref, ip_ref, x_ref, h1_ref, wa_ref, wb_ref,
             b1_ref, bb_ref, o_ref):
        in_c = jnp.maximum(ip_ref[0] + ip_ref[1], 1.0)
        dstn = lax.rsqrt(in_c)
        ninv = 1.0 / in_c
        agg = jnp.concatenate([p_ref[0], p_ref[1]], axis=-1)
        conv = agg * dstn + b1_ref[...]
        buf = (jnp.dot(x_ref[...], wa_ref[...],
                       preferred_element_type=jnp.float32)
               + jnp.dot(h1_ref[...], wb_ref[...],
                         preferred_element_type=jnp.float32) + bb_ref[...])
        o_ref[...] = conv + ninv * buf

    col = pl.BlockSpec((NC, RB, 1), lambda i: (0, i, 0))
    mat = pl.BlockSpec((RB, d), lambda i: (i, 0))
    wsp = pl.BlockSpec((d, d), lambda i: (0, 0))
    bsp = pl.BlockSpec((1, d), lambda i: (0, 0))
    psp = pl.BlockSpec((NC, RB, DH), lambda i: (0, i, 0))
    return pl.pallas_call(
        body,
        grid=(NPAD // RB,),
        in_specs=[psp, col, mat, mat, wsp, wsp, bsp, bsp],
        out_specs=mat,
        out_shape=jax.ShapeDtypeStruct((n, d), jnp.float32),
    )(parts, inp, x, h1, wb1a, wb1b, b1r, bb1r)


def kernel(features, edge_index, W0, b0, W1, b1, Wb0, bb0, Wb1, bb1):
    N, D = features.shape
    E = edge_index.shape[1]
    NPAD = N + PADR
    src = edge_index[0]
    dst = edge_index[1]

    CPW = -(-E // (NS * K))
    CPW = -(-CPW // NBUF) * NBUF
    E2 = NS * CPW * K
    pe = jnp.arange(E2 - E, dtype=jnp.int32)
    src_p = jnp.concatenate([src, pe % N]).reshape(NS, CPW, K)
    dst_p = jnp.concatenate([dst, N + pe % PADR]).reshape(NS, CPW, K)

    T = 2 * NPAD
    DE = 2 * E
    CD = -(-DE // (NW * K))
    CD = -(-CD // 8) * 8
    pd = jnp.arange(NW * CD * K - DE, dtype=jnp.int32)
    degidx = jnp.concatenate([dst, src + NPAD, N + pd % PADR])
    degidx = degidx.reshape(NW, CD, K)

    zeros_e = jnp.zeros((T // NS,), jnp.float32)
    zeros_r = jnp.zeros((K, DH), jnp.float32)

    degparts = _make_deg_kernel(T, CD)(degidx, zeros_e)
    dp = degparts.reshape(NC, 2, NPAD, 1)
    inp = dp[:, 0]
    outp = dp[:, 1]

    RB = 1024
    b0r, bb0r = b0.reshape(1, D), bb0.reshape(1, D)
    b1r, bb1r = b1.reshape(1, D), bb1.reshape(1, D)
    wb1a, wb1b = Wb1[:D], Wb1[D:]

    pass_fn = _make_pass_kernel(NPAD, CPW)

    feat0 = _mm_scale_call(features, W0, outp, NPAD, RB)
    parts0 = pass_fn(src_p, dst_p, feat0, zeros_r)
    h1, feat1 = _mid_call(parts0, inp, outp,
                          features, Wb0, b0r, bb0r, W1, NPAD, RB)

    parts1 = pass_fn(src_p, dst_p, feat1, zeros_r)
    out = _final_call(parts1, inp,
                      features, h1, wb1a, wb1b, b1r, bb1r, NPAD, RB)
    return out
